# baseline (speedup 1.0000x reference)
import functools

import jax
import jax.numpy as jnp
from jax.experimental import pallas as pl

_D = 64
_M = 512
_K = 8
_TAU = 0.1
_BLK = 512


def _body(q_ref, mem_ref, out_ref, smax_ref):
    q = q_ref[...]
    mem = mem_ref[...]

    qn = q * jax.lax.rsqrt(jnp.maximum(jnp.sum(q * q, axis=1, keepdims=True), 1e-24))
    mn = mem * jax.lax.rsqrt(jnp.maximum(jnp.sum(mem * mem, axis=1, keepdims=True), 1e-24))

    sim = jnp.dot(qn, mn.T, preferred_element_type=jnp.float32)

    neg = jnp.float32(-jnp.inf)
    smax = jnp.max(sim, axis=1, keepdims=True)
    thr = smax
    for _ in range(_K - 1):
        thr = jnp.max(jnp.where(sim < thr, sim, neg), axis=1, keepdims=True)

    e = jnp.where(sim >= thr, jnp.exp((sim - smax) * (1.0 / _TAU)), 0.0)
    denom = jnp.sum(e, axis=1, keepdims=True)

    r = jnp.dot(e, mem, preferred_element_type=jnp.float32)
    out_ref[...] = r * (1.0 / denom)
    smax_ref[...] = smax * (1.0 / _TAU)


@functools.partial(jax.jit, static_argnames=())
def kernel(queries, memory):
    b, n, d = queries.shape
    tokens = b * n
    q2 = queries.reshape(tokens, d)
    grid = (tokens // _BLK,)
    out, smax = pl.pallas_call(
        _body,
        grid=grid,
        in_specs=[
            pl.BlockSpec((_BLK, d), lambda i: (i, 0)),
            pl.BlockSpec((_M, d), lambda i: (0, 0)),
        ],
        out_specs=[
            pl.BlockSpec((_BLK, d), lambda i: (i, 0)),
            pl.BlockSpec((_BLK, 1), lambda i: (i, 0)),
        ],
        out_shape=[
            jax.ShapeDtypeStruct((tokens, d), jnp.float32),
            jax.ShapeDtypeStruct((tokens, 1), jnp.float32),
        ],
    )(q2, memory)
    return out.reshape(b, n, d), smax.reshape(b, n, 1)


# speedup vs baseline: 1.2391x; 1.2391x over previous
import functools

import jax
import jax.numpy as jnp
from jax.experimental import pallas as pl
from jax.experimental.pallas import tpu as pltpu

_D = 64
_M = 512
_K = 8
_TAU = 0.1
_SUB = 512
_NSUB = 2
_BLK = _SUB * _NSUB


def _body(q_ref, mem_ref, out_ref, smax_ref):
    mem = mem_ref[...]
    mn = mem * jax.lax.rsqrt(jnp.maximum(jnp.sum(mem * mem, axis=1, keepdims=True), 1e-24))

    for h in range(_NSUB):
        rows = pl.ds(h * _SUB, _SUB)
        q = q_ref[rows, :]
        qn = q * jax.lax.rsqrt(jnp.maximum(jnp.sum(q * q, axis=1, keepdims=True), 1e-24))

        sim = jnp.dot(qn, mn.T, preferred_element_type=jnp.float32)

        neg = jnp.float32(-jnp.inf)
        smax = jnp.max(sim, axis=1, keepdims=True)
        thr = smax
        for _ in range(_K - 1):
            thr = jnp.max(jnp.where(sim < thr, sim, neg), axis=1, keepdims=True)

        e = jnp.where(sim >= thr, jnp.exp((sim - smax) * (1.0 / _TAU)), 0.0)
        denom = jnp.sum(e, axis=1, keepdims=True)

        r = jnp.dot(e, mem, preferred_element_type=jnp.float32)
        out_ref[rows, :] = r * (1.0 / denom)
        smax_ref[rows, :] = smax * (1.0 / _TAU)


@functools.partial(jax.jit, static_argnames=())
def kernel(queries, memory):
    b, n, d = queries.shape
    tokens = b * n
    q2 = queries.reshape(tokens, d)
    grid = (tokens // _BLK,)
    out, smax = pl.pallas_call(
        _body,
        grid=grid,
        in_specs=[
            pl.BlockSpec((_BLK, d), lambda i: (i, 0)),
            pl.BlockSpec((_M, d), lambda i: (0, 0)),
        ],
        out_specs=[
            pl.BlockSpec((_BLK, d), lambda i: (i, 0)),
            pl.BlockSpec((_BLK, 1), lambda i: (i, 0)),
        ],
        out_shape=[
            jax.ShapeDtypeStruct((tokens, d), jnp.float32),
            jax.ShapeDtypeStruct((tokens, 1), jnp.float32),
        ],
        compiler_params=pltpu.CompilerParams(
            dimension_semantics=("parallel",),
        ),
    )(q2, memory)
    return out.reshape(b, n, d), smax.reshape(b, n, 1)


# speedup vs baseline: 1.2825x; 1.0350x over previous
import functools

import jax
import jax.numpy as jnp
from jax.experimental import pallas as pl
from jax.experimental.pallas import tpu as pltpu

_D = 64
_M = 512
_K = 8
_TAU = 0.1
_SUB = 512
_NSUB = 4
_BLK = _SUB * _NSUB


def _body(q_ref, mem_ref, out_ref, smax_ref):
    mem = mem_ref[...]
    mn = mem * jax.lax.rsqrt(jnp.maximum(jnp.sum(mem * mem, axis=1, keepdims=True), 1e-24))

    for h in range(_NSUB):
        rows = pl.ds(h * _SUB, _SUB)
        q = q_ref[rows, :]
        qn = q * jax.lax.rsqrt(jnp.maximum(jnp.sum(q * q, axis=1, keepdims=True), 1e-24))

        sim = jnp.dot(qn, mn.T, preferred_element_type=jnp.float32)

        neg = jnp.float32(-jnp.inf)
        smax = jnp.max(sim, axis=1, keepdims=True)
        thr = smax
        for _ in range(_K - 1):
            thr = jnp.max(jnp.where(sim < thr, sim, neg), axis=1, keepdims=True)

        smax_s = smax * (1.0 / _TAU)
        e = jnp.where(sim >= thr, jnp.exp(sim * (1.0 / _TAU) - smax_s), 0.0)
        denom = jnp.sum(e, axis=1, keepdims=True)

        r = jnp.dot(e, mem, preferred_element_type=jnp.float32)
        out_ref[rows, :] = r * (1.0 / denom)
        smax_ref[rows, :] = smax_s


@functools.partial(jax.jit, static_argnames=())
def kernel(queries, memory):
    b, n, d = queries.shape
    tokens = b * n
    q2 = queries.reshape(tokens, d)
    grid = (tokens // _BLK,)
    out, smax = pl.pallas_call(
        _body,
        grid=grid,
        in_specs=[
            pl.BlockSpec((_BLK, d), lambda i: (i, 0)),
            pl.BlockSpec((_M, d), lambda i: (0, 0)),
        ],
        out_specs=[
            pl.BlockSpec((_BLK, d), lambda i: (i, 0)),
            pl.BlockSpec((_BLK, 1), lambda i: (i, 0)),
        ],
        out_shape=[
            jax.ShapeDtypeStruct((tokens, d), jnp.float32),
            jax.ShapeDtypeStruct((tokens, 1), jnp.float32),
        ],
        compiler_params=pltpu.CompilerParams(
            dimension_semantics=("parallel",),
        ),
    )(q2, memory)
    return out.reshape(b, n, d), smax.reshape(b, n, 1)
